# Initial kernel scaffold; baseline (speedup 1.0000x reference)
import jax
import jax.numpy as jnp
from jax.experimental import pallas as pl


def kernel(inptxtFeats, inpCoords, neigh_vert, neigh_hor, token_table, pos_table, W_prior, b_prior, W_emb, b_emb, W_prior1, b_prior1, W_emb1, b_emb1):
    raise NotImplementedError("write your pallas kernel here")


# speedup vs baseline: 3.3757x; 3.3757x over previous
import functools

import jax
import jax.numpy as jnp
from jax import lax
from jax.experimental import pallas as pl
from jax.experimental.pallas import tpu as pltpu
from jax.experimental.pallas import tpu_sc as plsc

B = 1024
L = 200
D = 128
NEIGH = 256
V = 100000
M = B * L


_NC = 2
_NS = 16
_NW = _NC * _NS
_PW = M // _NW
_G = 128
_NG = _PW // _G


def _sc_gather_body(table_hbm, idx_hbm, out_hbm, idx_v, rows_v, sem):
    wid = lax.axis_index("s") * _NC + lax.axis_index("c")
    base = wid * _PW
    pltpu.sync_copy(idx_hbm.at[wid], idx_v)

    def body(j, carry):
        pltpu.async_copy(table_hbm.at[idx_v.at[j]], rows_v, sem).wait()
        pltpu.sync_copy(rows_v, out_hbm.at[pl.ds(base + j * _G, _G)])
        return carry

    lax.fori_loop(0, _NG, body, 0, unroll=False)


def _sc_gather(token_table, idx_flat):
    mesh = plsc.VectorSubcoreMesh(core_axis_name="c", subcore_axis_name="s")
    idx3 = idx_flat.reshape(_NW, _NG, _G)
    return pl.kernel(
        _sc_gather_body,
        out_type=jax.ShapeDtypeStruct((M, D), jnp.float32),
        mesh=mesh,
        scratch_types=[
            pltpu.VMEM((_NG, _G), jnp.int32),
            pltpu.VMEM((_G, D), jnp.float32),
            pltpu.SemaphoreType.DMA,
        ],
    )(token_table, idx3)


_BB = 16


def _tc_body(nv_ref, nh_ref, tok_ref, pos_ref, wp_ref, bp_ref, we_ref, be_ref,
             wp1_ref, bp1_ref, we1_ref, be1_ref, out_ref, wv_s, wh_s, pb_s):
    @pl.when(pl.program_id(0) == 0)
    def _fold():
        wv_s[...] = jnp.dot(wp_ref[...], we_ref[...],
                            preferred_element_type=jnp.float32)
        wh_s[...] = jnp.dot(wp1_ref[...], we1_ref[...],
                            preferred_element_type=jnp.float32)
        bias = (jnp.dot(bp_ref[...], we_ref[...],
                        preferred_element_type=jnp.float32)
                + be_ref[...]
                + jnp.dot(bp1_ref[...], we1_ref[...],
                          preferred_element_type=jnp.float32)
                + be1_ref[...])
        pb_s[...] = pos_ref[...] + bias

    xv = nv_ref[...].reshape(_BB * L, NEIGH)
    xh = nh_ref[...].reshape(_BB * L, NEIGH)
    acc = jnp.dot(xv, wv_s[...], preferred_element_type=jnp.float32)
    acc = acc + jnp.dot(xh, wh_s[...], preferred_element_type=jnp.float32)
    out_ref[...] = acc.reshape(_BB, L, D) + tok_ref[...] + pb_s[...][None, :, :]


def _tc_dense(nv, nh, tok, pos_table, W_prior, b_prior, W_emb, b_emb,
              W_prior1, b_prior1, W_emb1, b_emb1):
    grid = (B // _BB,)
    blk = lambda i: (i, 0, 0)
    rep2 = lambda i: (0, 0)
    return pl.pallas_call(
        _tc_body,
        grid=grid,
        in_specs=[
            pl.BlockSpec((_BB, L, NEIGH), blk),
            pl.BlockSpec((_BB, L, NEIGH), blk),
            pl.BlockSpec((_BB, L, D), blk),
            pl.BlockSpec((L, D), rep2),
            pl.BlockSpec((NEIGH, D), rep2),
            pl.BlockSpec((1, D), rep2),
            pl.BlockSpec((D, D), rep2),
            pl.BlockSpec((1, D), rep2),
            pl.BlockSpec((NEIGH, D), rep2),
            pl.BlockSpec((1, D), rep2),
            pl.BlockSpec((D, D), rep2),
            pl.BlockSpec((1, D), rep2),
        ],
        out_specs=pl.BlockSpec((_BB, L, D), blk),
        out_shape=jax.ShapeDtypeStruct((B, L, D), jnp.float32),
        scratch_shapes=[
            pltpu.VMEM((NEIGH, D), jnp.float32),
            pltpu.VMEM((NEIGH, D), jnp.float32),
            pltpu.VMEM((L, D), jnp.float32),
        ],
    )(nv, nh, tok, pos_table, W_prior, b_prior.reshape(1, D), W_emb,
      b_emb.reshape(1, D), W_prior1, b_prior1.reshape(1, D), W_emb1,
      b_emb1.reshape(1, D))


def kernel(inptxtFeats, inpCoords, neigh_vert, neigh_hor, token_table,
           pos_table, W_prior, b_prior, W_emb, b_emb, W_prior1, b_prior1,
           W_emb1, b_emb1):
    del inpCoords
    idx_flat = inptxtFeats.reshape(M)
    tok = _sc_gather(token_table, idx_flat).reshape(B, L, D)
    return _tc_dense(neigh_vert, neigh_hor, tok, pos_table, W_prior, b_prior,
                     W_emb, b_emb, W_prior1, b_prior1, W_emb1, b_emb1)
